# baseline (speedup 1.0000x reference)
import functools

import jax
import jax.numpy as jnp
from jax import lax
from jax.experimental import pallas as pl
from jax.experimental.pallas import tpu as pltpu
from jax.experimental.pallas import tpu_sc as plsc

NN = 10000
EE = 320000
GG = 64
SLOPE = 0.2
CHUNK = 16
NBUF = 5

_f32 = jnp.float32


def _lrelu(v):
    return jnp.maximum(v, SLOPE * v)


def _prep_body(x_ref, w_ref, ms_ref, md_ref, h_ref, ts_ref, td_ref, gm_ref,
               exl_ref):
    h = jnp.dot(x_ref[...], w_ref[...], preferred_element_type=_f32,
                precision=lax.Precision.HIGHEST)
    h_ref[...] = h
    a_s = jnp.dot(h, ms_ref[...], preferred_element_type=_f32,
                  precision=lax.Precision.HIGHEST)
    a_d = jnp.dot(h, md_ref[...], preferred_element_type=_f32,
                  precision=lax.Precision.HIGHEST)
    ts_ref[...] = a_s
    td_ref[...] = a_d
    gm = jnp.max(a_s, axis=0, keepdims=True)
    gm_ref[...] = gm
    b = _lrelu(gm + a_d)
    exl_ref[...] = jnp.exp(_lrelu(a_s + a_d) - b)


def _prep(x, w, ms, md):
    return pl.pallas_call(
        _prep_body,
        out_shape=[
            jax.ShapeDtypeStruct((NN, 128), _f32),
            jax.ShapeDtypeStruct((NN, 16), _f32),
            jax.ShapeDtypeStruct((NN, 16), _f32),
            jax.ShapeDtypeStruct((1, 16), _f32),
            jax.ShapeDtypeStruct((NN, 16), _f32),
        ],
    )(x, w, ms, md)


def _head_expander():
    row = lax.broadcasted_iota(jnp.int32, (8, 128), 0)
    col = lax.broadcasted_iota(jnp.int32, (8, 128), 1)
    return (col // 16 == row).astype(_f32)


_MB = 2000


def _mid_body(po_ref, pd_ref, exl_ref, h1_ref, b1_ref, w2_ref, ms2_ref,
              md2_ref, h2_ref, ts2_ref, td2_ref, gm2_ref):
    rep = _head_expander()
    exl = exl_ref[...]
    den8 = (pd_ref[0] + pd_ref[1] + exl)[:, :8]
    h1 = h1_ref[...]
    num = po_ref[0] + po_ref[1] + jnp.dot(
        exl[:, :8], rep, preferred_element_type=_f32) * h1
    dinv = 1.0 / jnp.dot(den8, rep, preferred_element_type=_f32)
    h1o = num * dinv + b1_ref[...]
    e = jnp.where(h1o > 0, h1o, jnp.exp(h1o) - 1.0)
    h2 = jnp.dot(e, w2_ref[...], preferred_element_type=_f32,
                 precision=lax.Precision.HIGHEST)
    h2_ref[...] = h2
    a_s = jnp.dot(h2, ms2_ref[...], preferred_element_type=_f32,
                  precision=lax.Precision.HIGHEST)
    a_d = jnp.dot(h2, md2_ref[...], preferred_element_type=_f32,
                  precision=lax.Precision.HIGHEST)
    ts2_ref[...] = a_s
    td2_ref[...] = a_d

    @pl.when(pl.program_id(0) == 0)
    def _():
        gm2_ref[...] = jnp.full((1, 16), -jnp.inf, _f32)

    gm2_ref[...] = jnp.maximum(gm2_ref[...],
                               jnp.max(a_s, axis=0, keepdims=True))


def _mid(po, pd, exl, h1, b1, w2, ms2, md2):
    nb = NN // _MB
    return pl.pallas_call(
        _mid_body,
        grid=(nb,),
        in_specs=[
            pl.BlockSpec((2, _MB, 128), lambda i: (0, i, 0)),
            pl.BlockSpec((2, _MB, 16), lambda i: (0, i, 0)),
            pl.BlockSpec((_MB, 16), lambda i: (i, 0)),
            pl.BlockSpec((_MB, 128), lambda i: (i, 0)),
            pl.BlockSpec((1, 128), lambda i: (0, 0)),
            pl.BlockSpec((128, 128), lambda i: (0, 0)),
            pl.BlockSpec((128, 16), lambda i: (0, 0)),
            pl.BlockSpec((128, 16), lambda i: (0, 0)),
        ],
        out_specs=[
            pl.BlockSpec((_MB, 128), lambda i: (i, 0)),
            pl.BlockSpec((_MB, 16), lambda i: (i, 0)),
            pl.BlockSpec((_MB, 16), lambda i: (i, 0)),
            pl.BlockSpec((1, 16), lambda i: (0, 0)),
        ],
        out_shape=[
            jax.ShapeDtypeStruct((NN, 128), _f32),
            jax.ShapeDtypeStruct((NN, 16), _f32),
            jax.ShapeDtypeStruct((NN, 16), _f32),
            jax.ShapeDtypeStruct((1, 16), _f32),
        ],
    )(po, pd, exl, h1, b1, w2, ms2, md2)


def _final_body(po_ref, pd_ref, ts2_ref, td2_ref, gm2_ref, h2_ref, b2_ref,
                batch_ref, g_ref):
    a_s = ts2_ref[...]
    a_d = td2_ref[...]
    gm = gm2_ref[...]
    exl2 = jnp.exp(_lrelu(a_s + a_d) - _lrelu(gm + a_d))
    num = po_ref[0] + po_ref[1] + exl2[:, 0:1] * h2_ref[...]
    den = (pd_ref[0] + pd_ref[1] + exl2)[:, 0:1]
    out2 = num / den + b2_ref[...]
    bi = batch_ref[...]
    gi = lax.broadcasted_iota(jnp.int32, (GG, NN), 0)
    oh = (bi == gi).astype(_f32)
    sums = jnp.dot(oh, out2, preferred_element_type=_f32,
                   precision=lax.Precision.HIGHEST)
    counts = jnp.sum(oh, axis=1, keepdims=True)
    g_ref[...] = sums / jnp.maximum(counts, 1.0)


def _final(po, pd, ts2, td2, gm2, h2, b2, batch2d):
    return pl.pallas_call(
        _final_body,
        out_shape=jax.ShapeDtypeStruct((GG, 128), _f32),
    )(po, pd, ts2, td2, gm2, h2, b2, batch2d)


def _make_edge_kernel(nheads):
    mesh = plsc.VectorSubcoreMesh(core_axis_name="c", subcore_axis_name="s")
    ept = EE // 32
    nchunks = ept // CHUNK
    nrows = 624

    @functools.partial(
        pl.kernel,
        out_type=[
            jax.ShapeDtypeStruct((2, NN, 16), _f32),
            jax.ShapeDtypeStruct((2, NN, 128), _f32),
        ],
        mesh=mesh,
        compiler_params=pltpu.CompilerParams(use_tc_tiling_on_sc=False),
        scratch_types=[
            pltpu.VMEM((EE // 32 // CHUNK, CHUNK), jnp.int32),
            pltpu.VMEM((EE // 32 // CHUNK, CHUNK), jnp.int32),
            [pltpu.VMEM((CHUNK, 16), _f32) for _ in range(NBUF)],
            [pltpu.VMEM((CHUNK, 16), _f32) for _ in range(NBUF)],
            [pltpu.VMEM((CHUNK, 128), _f32) for _ in range(NBUF)],
            [pltpu.VMEM((CHUNK, 16), _f32) for _ in range(NBUF)],
            pltpu.VMEM((16,), _f32),
            pltpu.VMEM_SHARED((NN, 16), _f32),
            pltpu.VMEM_SHARED((NN, 128), _f32),
            [pltpu.SemaphoreType.DMA for _ in range(NBUF)],
            [pltpu.SemaphoreType.DMA for _ in range(NBUF)],
        ],
    )
    def edge_kernel(src_h, dst_h, ts_h, td_h, h_h, gm_h, z128_h, z16_h,
                    den_o, out_o, idxs_v, idxd_v, rs_v, rd_v, hb_v, exb_v,
                    gm_v, den_sh, out_sh, semg, sems):
        cid = lax.axis_index("c")
        sid = lax.axis_index("s")
        wid = sid * 2 + cid
        r0 = sid * nrows
        pltpu.sync_copy(z128_h.at[pl.ds(r0, nrows)],
                        out_sh.at[pl.ds(r0, nrows)])
        pltpu.sync_copy(z16_h.at[pl.ds(r0, nrows)],
                        den_sh.at[pl.ds(r0, nrows)])

        @pl.when(sid == 15)
        def _():
            tail = 16 * nrows
            pltpu.sync_copy(z128_h.at[pl.ds(tail, NN - tail)],
                            out_sh.at[pl.ds(tail, NN - tail)])
            pltpu.sync_copy(z16_h.at[pl.ds(tail, NN - tail)],
                            den_sh.at[pl.ds(tail, NN - tail)])

        pltpu.sync_copy(gm_h, gm_v)
        rbase = wid * nchunks
        pltpu.sync_copy(src_h.at[pl.ds(rbase, nchunks)], idxs_v)
        pltpu.sync_copy(dst_h.at[pl.ds(rbase, nchunks)], idxd_v)
        plsc.subcore_barrier()

        def fire_gather(j, b):
            pltpu.async_copy(ts_h.at[idxs_v.at[j]], rs_v[b], semg[b])
            pltpu.async_copy(td_h.at[idxd_v.at[j]], rd_v[b], semg[b])
            pltpu.async_copy(h_h.at[idxs_v.at[j]], hb_v[b], semg[b])

        def wait_gather(b):
            pltpu.make_async_copy(ts_h.at[idxs_v.at[0]], rs_v[b],
                                  semg[b]).wait()
            pltpu.make_async_copy(td_h.at[idxd_v.at[0]], rd_v[b],
                                  semg[b]).wait()
            pltpu.make_async_copy(h_h.at[idxs_v.at[0]], hb_v[b],
                                  semg[b]).wait()

        def compute(b):
            gm = gm_v[...]

            @plsc.parallel_loop(0, CHUNK, unroll=4)
            def edge(i):
                rs = rs_v[b][i, :]
                rd = rd_v[b][i, :]
                al = _lrelu(rs + rd)
                bb = _lrelu(gm + rd)
                ex = jnp.exp(al - bb)
                exb_v[b][i, :] = ex
                for hd in range(8):
                    lane = hd if nheads == 8 else 0
                    sc = ex[lane]
                    sl = pl.ds(hd * 16, 16)
                    hb_v[b][i, sl] = hb_v[b][i, sl] * sc

        def fire_scatter(j, b):
            pltpu.async_copy(exb_v[b], den_sh.at[idxd_v.at[j]], sems[b],
                             add=True)
            pltpu.async_copy(hb_v[b], out_sh.at[idxd_v.at[j]], sems[b],
                             add=True)

        def wait_scatter(b):
            pltpu.make_async_copy(exb_v[b], den_sh.at[idxd_v.at[0]],
                                  sems[b]).wait()
            pltpu.make_async_copy(hb_v[b], out_sh.at[idxd_v.at[0]],
                                  sems[b]).wait()

        LOOKAHEAD = 3

        def block5(jj, first, last):
            for b in range(NBUF):
                j = jj * NBUF + b
                wait_gather(b)
                compute(b)
                fire_scatter(j, b)
                bb = (b + LOOKAHEAD) % NBUF
                if not (first and b < LOOKAHEAD - 1):
                    wait_scatter(bb)
                if not last or b < NBUF - LOOKAHEAD:
                    fire_gather(j + LOOKAHEAD, bb)

        nblocks = nchunks // NBUF
        for b in range(LOOKAHEAD):
            fire_gather(b, b)
        block5(0, True, False)

        def body(jj, carry):
            block5(jj, False, False)
            return carry

        lax.fori_loop(1, nblocks - 1, body, 0)
        block5(nblocks - 1, False, True)
        wait_scatter(LOOKAHEAD)
        wait_scatter(LOOKAHEAD + 1)
        plsc.subcore_barrier()
        pltpu.sync_copy(den_sh.at[pl.ds(r0, nrows)],
                        den_o.at[cid, pl.ds(r0, nrows)])
        pltpu.sync_copy(out_sh.at[pl.ds(r0, nrows)],
                        out_o.at[cid, pl.ds(r0, nrows)])

        @pl.when(sid == 15)
        def _():
            tail = 16 * nrows
            pltpu.sync_copy(den_sh.at[pl.ds(tail, NN - tail)],
                            den_o.at[cid, pl.ds(tail, NN - tail)])
            pltpu.sync_copy(out_sh.at[pl.ds(tail, NN - tail)],
                            out_o.at[cid, pl.ds(tail, NN - tail)])

    return edge_kernel


@functools.lru_cache(maxsize=None)
def _edge_kernel(nheads):
    return _make_edge_kernel(nheads)


def _edge8(*args):
    return _edge_kernel(8)(*args)


def _edge1(*args):
    return _edge_kernel(1)(*args)


def _head_proj(att, heads, hid):
    a = att.reshape(heads, hid).astype(_f32)
    eye = jnp.eye(16, dtype=_f32)[:heads]
    return (a[:, :, None] * eye[:, None, :]).reshape(heads * hid, 16)


def kernel(x, edge_index, batch, W1, att_src1, att_dst1, b1, W2, att_src2,
           att_dst2, b2):
    src = edge_index[0].astype(jnp.int32)
    dst = edge_index[1].astype(jnp.int32)
    ms1 = _head_proj(att_src1, 8, 16)
    md1 = _head_proj(att_dst1, 8, 16)
    ms2 = _head_proj(att_src2, 1, 128)
    md2 = _head_proj(att_dst2, 1, 128)
    z128 = jnp.zeros((NN, 128), _f32)
    z16 = jnp.zeros((NN, 16), _f32)

    src = src.reshape(-1, CHUNK)
    dst = dst.reshape(-1, CHUNK)
    h1, ts1, td1, gm1, exl1 = _prep(x, W1.astype(_f32), ms1, md1)
    pd1, po1 = _edge8(src, dst, ts1, td1, h1, gm1.reshape(16), z128, z16)
    h2, ts2, td2, gm2 = _mid(po1, pd1, exl1, h1,
                             b1.reshape(1, 128).astype(_f32),
                             W2.astype(_f32), ms2, md2)
    pd2, po2 = _edge1(src, dst, ts2, td2, h2, gm2.reshape(16), z128, z16)
    g = _final(po2, pd2, ts2, td2, gm2, h2,
               b2.reshape(1, 128).astype(_f32),
               batch.reshape(1, NN).astype(jnp.int32))
    return g


# speedup vs baseline: 1.0405x; 1.0405x over previous
import functools

import jax
import jax.numpy as jnp
from jax import lax
from jax.experimental import pallas as pl
from jax.experimental.pallas import tpu as pltpu
from jax.experimental.pallas import tpu_sc as plsc

NN = 10000
EE = 320000
GG = 64
SLOPE = 0.2
CHUNK = 16
NBUF = 5

_f32 = jnp.float32


def _lrelu(v):
    return jnp.maximum(v, SLOPE * v)


def _prep_body(x_ref, w_ref, ms_ref, md_ref, h_ref, ts_ref, td_ref, gm_ref,
               exl_ref):
    h = jnp.dot(x_ref[...], w_ref[...], preferred_element_type=_f32,
                precision=lax.Precision.DEFAULT)
    h_ref[...] = h
    a_s = jnp.dot(h, ms_ref[...], preferred_element_type=_f32,
                  precision=lax.Precision.DEFAULT)
    a_d = jnp.dot(h, md_ref[...], preferred_element_type=_f32,
                  precision=lax.Precision.DEFAULT)
    ts_ref[...] = a_s
    td_ref[...] = a_d
    gm = jnp.max(a_s, axis=0, keepdims=True)
    gm_ref[...] = gm
    b = _lrelu(gm + a_d)
    exl_ref[...] = jnp.exp(_lrelu(a_s + a_d) - b)


def _prep(x, w, ms, md):
    return pl.pallas_call(
        _prep_body,
        out_shape=[
            jax.ShapeDtypeStruct((NN, 128), _f32),
            jax.ShapeDtypeStruct((NN, 16), _f32),
            jax.ShapeDtypeStruct((NN, 16), _f32),
            jax.ShapeDtypeStruct((1, 16), _f32),
            jax.ShapeDtypeStruct((NN, 16), _f32),
        ],
    )(x, w, ms, md)


def _head_expander():
    row = lax.broadcasted_iota(jnp.int32, (8, 128), 0)
    col = lax.broadcasted_iota(jnp.int32, (8, 128), 1)
    return (col // 16 == row).astype(_f32)


_MB = 2000


def _mid_body(po_ref, pd_ref, exl_ref, h1_ref, b1_ref, w2_ref, ms2_ref,
              md2_ref, h2_ref, ts2_ref, td2_ref, gm2_ref):
    rep = _head_expander()
    exl = exl_ref[...]
    den8 = (pd_ref[0] + pd_ref[1] + exl)[:, :8]
    h1 = h1_ref[...]
    num = po_ref[0] + po_ref[1] + jnp.dot(
        exl[:, :8], rep, preferred_element_type=_f32) * h1
    dinv = 1.0 / jnp.dot(den8, rep, preferred_element_type=_f32)
    h1o = num * dinv + b1_ref[...]
    e = jnp.where(h1o > 0, h1o, jnp.exp(h1o) - 1.0)
    h2 = jnp.dot(e, w2_ref[...], preferred_element_type=_f32,
                 precision=lax.Precision.DEFAULT)
    h2_ref[...] = h2
    a_s = jnp.dot(h2, ms2_ref[...], preferred_element_type=_f32,
                  precision=lax.Precision.DEFAULT)
    a_d = jnp.dot(h2, md2_ref[...], preferred_element_type=_f32,
                  precision=lax.Precision.DEFAULT)
    ts2_ref[...] = a_s
    td2_ref[...] = a_d

    @pl.when(pl.program_id(0) == 0)
    def _():
        gm2_ref[...] = jnp.full((1, 16), -jnp.inf, _f32)

    gm2_ref[...] = jnp.maximum(gm2_ref[...],
                               jnp.max(a_s, axis=0, keepdims=True))


def _mid(po, pd, exl, h1, b1, w2, ms2, md2):
    nb = NN // _MB
    return pl.pallas_call(
        _mid_body,
        grid=(nb,),
        in_specs=[
            pl.BlockSpec((2, _MB, 128), lambda i: (0, i, 0)),
            pl.BlockSpec((2, _MB, 16), lambda i: (0, i, 0)),
            pl.BlockSpec((_MB, 16), lambda i: (i, 0)),
            pl.BlockSpec((_MB, 128), lambda i: (i, 0)),
            pl.BlockSpec((1, 128), lambda i: (0, 0)),
            pl.BlockSpec((128, 128), lambda i: (0, 0)),
            pl.BlockSpec((128, 16), lambda i: (0, 0)),
            pl.BlockSpec((128, 16), lambda i: (0, 0)),
        ],
        out_specs=[
            pl.BlockSpec((_MB, 128), lambda i: (i, 0)),
            pl.BlockSpec((_MB, 16), lambda i: (i, 0)),
            pl.BlockSpec((_MB, 16), lambda i: (i, 0)),
            pl.BlockSpec((1, 16), lambda i: (0, 0)),
        ],
        out_shape=[
            jax.ShapeDtypeStruct((NN, 128), _f32),
            jax.ShapeDtypeStruct((NN, 16), _f32),
            jax.ShapeDtypeStruct((NN, 16), _f32),
            jax.ShapeDtypeStruct((1, 16), _f32),
        ],
    )(po, pd, exl, h1, b1, w2, ms2, md2)


def _final_body(po_ref, pd_ref, ts2_ref, td2_ref, gm2_ref, h2_ref, b2_ref,
                batch_ref, g_ref):
    a_s = ts2_ref[...]
    a_d = td2_ref[...]
    gm = gm2_ref[...]
    exl2 = jnp.exp(_lrelu(a_s + a_d) - _lrelu(gm + a_d))
    num = po_ref[0] + po_ref[1] + exl2[:, 0:1] * h2_ref[...]
    den = (pd_ref[0] + pd_ref[1] + exl2)[:, 0:1]
    out2 = num / den + b2_ref[...]
    bi = batch_ref[...]
    gi = lax.broadcasted_iota(jnp.int32, (GG, NN), 0)
    oh = (bi == gi).astype(_f32)
    sums = jnp.dot(oh, out2, preferred_element_type=_f32,
                   precision=lax.Precision.DEFAULT)
    counts = jnp.sum(oh, axis=1, keepdims=True)
    g_ref[...] = sums / jnp.maximum(counts, 1.0)


def _final(po, pd, ts2, td2, gm2, h2, b2, batch2d):
    return pl.pallas_call(
        _final_body,
        out_shape=jax.ShapeDtypeStruct((GG, 128), _f32),
    )(po, pd, ts2, td2, gm2, h2, b2, batch2d)


def _make_edge_kernel(nheads):
    mesh = plsc.VectorSubcoreMesh(core_axis_name="c", subcore_axis_name="s")
    ept = EE // 32
    nchunks = ept // CHUNK
    nrows = 624

    @functools.partial(
        pl.kernel,
        out_type=[
            jax.ShapeDtypeStruct((2, NN, 16), _f32),
            jax.ShapeDtypeStruct((2, NN, 128), _f32),
        ],
        mesh=mesh,
        compiler_params=pltpu.CompilerParams(use_tc_tiling_on_sc=False),
        scratch_types=[
            pltpu.VMEM((EE // 32 // CHUNK, CHUNK), jnp.int32),
            pltpu.VMEM((EE // 32 // CHUNK, CHUNK), jnp.int32),
            [pltpu.VMEM((CHUNK, 16), _f32) for _ in range(NBUF)],
            [pltpu.VMEM((CHUNK, 16), _f32) for _ in range(NBUF)],
            [pltpu.VMEM((CHUNK, 128), _f32) for _ in range(NBUF)],
            [pltpu.VMEM((CHUNK, 16), _f32) for _ in range(NBUF)],
            pltpu.VMEM((16,), _f32),
            pltpu.VMEM_SHARED((NN, 16), _f32),
            pltpu.VMEM_SHARED((NN, 128), _f32),
            [pltpu.SemaphoreType.DMA for _ in range(NBUF)],
            [pltpu.SemaphoreType.DMA for _ in range(NBUF)],
        ],
    )
    def edge_kernel(src_h, dst_h, ts_h, td_h, h_h, gm_h, z128_h, z16_h,
                    den_o, out_o, idxs_v, idxd_v, rs_v, rd_v, hb_v, exb_v,
                    gm_v, den_sh, out_sh, semg, sems):
        cid = lax.axis_index("c")
        sid = lax.axis_index("s")
        wid = sid * 2 + cid
        r0 = sid * nrows
        pltpu.sync_copy(z128_h.at[pl.ds(r0, nrows)],
                        out_sh.at[pl.ds(r0, nrows)])
        pltpu.sync_copy(z16_h.at[pl.ds(r0, nrows)],
                        den_sh.at[pl.ds(r0, nrows)])

        @pl.when(sid == 15)
        def _():
            tail = 16 * nrows
            pltpu.sync_copy(z128_h.at[pl.ds(tail, NN - tail)],
                            out_sh.at[pl.ds(tail, NN - tail)])
            pltpu.sync_copy(z16_h.at[pl.ds(tail, NN - tail)],
                            den_sh.at[pl.ds(tail, NN - tail)])

        pltpu.sync_copy(gm_h, gm_v)
        rbase = wid * nchunks
        pltpu.sync_copy(src_h.at[pl.ds(rbase, nchunks)], idxs_v)
        pltpu.sync_copy(dst_h.at[pl.ds(rbase, nchunks)], idxd_v)
        plsc.subcore_barrier()

        def fire_gather(j, b):
            pltpu.async_copy(ts_h.at[idxs_v.at[j]], rs_v[b], semg[b])
            pltpu.async_copy(td_h.at[idxd_v.at[j]], rd_v[b], semg[b])
            pltpu.async_copy(h_h.at[idxs_v.at[j]], hb_v[b], semg[b])

        def wait_gather(b):
            pltpu.make_async_copy(ts_h.at[idxs_v.at[0]], rs_v[b],
                                  semg[b]).wait()
            pltpu.make_async_copy(td_h.at[idxd_v.at[0]], rd_v[b],
                                  semg[b]).wait()
            pltpu.make_async_copy(h_h.at[idxs_v.at[0]], hb_v[b],
                                  semg[b]).wait()

        def compute(b):
            gm = gm_v[...]

            @plsc.parallel_loop(0, CHUNK, unroll=2)
            def edge(i):
                rs = rs_v[b][i, :]
                rd = rd_v[b][i, :]
                al = _lrelu(rs + rd)
                bb = _lrelu(gm + rd)
                ex = jnp.exp(al - bb)
                exb_v[b][i, :] = ex
                for hd in range(8):
                    lane = hd if nheads == 8 else 0
                    sc = ex[lane]
                    sl = pl.ds(hd * 16, 16)
                    hb_v[b][i, sl] = hb_v[b][i, sl] * sc

        def fire_scatter(j, b):
            pltpu.async_copy(exb_v[b], den_sh.at[idxd_v.at[j]], sems[b],
                             add=True)
            pltpu.async_copy(hb_v[b], out_sh.at[idxd_v.at[j]], sems[b],
                             add=True)

        def wait_scatter(b):
            pltpu.make_async_copy(exb_v[b], den_sh.at[idxd_v.at[0]],
                                  sems[b]).wait()
            pltpu.make_async_copy(hb_v[b], out_sh.at[idxd_v.at[0]],
                                  sems[b]).wait()

        LOOKAHEAD = 3

        def block5(jj, first, last):
            for b in range(NBUF):
                j = jj * NBUF + b
                wait_gather(b)
                compute(b)
                fire_scatter(j, b)
                bb = (b + LOOKAHEAD) % NBUF
                if not (first and b < LOOKAHEAD - 1):
                    wait_scatter(bb)
                if not last or b < NBUF - LOOKAHEAD:
                    fire_gather(j + LOOKAHEAD, bb)

        nblocks = nchunks // NBUF
        for b in range(LOOKAHEAD):
            fire_gather(b, b)
        block5(0, True, False)

        def body(jj, carry):
            block5(jj, False, False)
            return carry

        lax.fori_loop(1, nblocks - 1, body, 0)
        block5(nblocks - 1, False, True)
        wait_scatter(LOOKAHEAD)
        wait_scatter(LOOKAHEAD + 1)
        plsc.subcore_barrier()
        pltpu.sync_copy(den_sh.at[pl.ds(r0, nrows)],
                        den_o.at[cid, pl.ds(r0, nrows)])
        pltpu.sync_copy(out_sh.at[pl.ds(r0, nrows)],
                        out_o.at[cid, pl.ds(r0, nrows)])

        @pl.when(sid == 15)
        def _():
            tail = 16 * nrows
            pltpu.sync_copy(den_sh.at[pl.ds(tail, NN - tail)],
                            den_o.at[cid, pl.ds(tail, NN - tail)])
            pltpu.sync_copy(out_sh.at[pl.ds(tail, NN - tail)],
                            out_o.at[cid, pl.ds(tail, NN - tail)])

    return edge_kernel


@functools.lru_cache(maxsize=None)
def _edge_kernel(nheads):
    return _make_edge_kernel(nheads)


def _edge8(*args):
    return _edge_kernel(8)(*args)


def _edge1(*args):
    return _edge_kernel(1)(*args)


def _head_proj(att, heads, hid):
    a = att.reshape(heads, hid).astype(_f32)
    eye = jnp.eye(16, dtype=_f32)[:heads]
    return (a[:, :, None] * eye[:, None, :]).reshape(heads * hid, 16)


def kernel(x, edge_index, batch, W1, att_src1, att_dst1, b1, W2, att_src2,
           att_dst2, b2):
    src = edge_index[0].astype(jnp.int32)
    dst = edge_index[1].astype(jnp.int32)
    ms1 = _head_proj(att_src1, 8, 16)
    md1 = _head_proj(att_dst1, 8, 16)
    ms2 = _head_proj(att_src2, 1, 128)
    md2 = _head_proj(att_dst2, 1, 128)
    z128 = jnp.zeros((NN, 128), _f32)
    z16 = jnp.zeros((NN, 16), _f32)

    src = src.reshape(-1, CHUNK)
    dst = dst.reshape(-1, CHUNK)
    h1, ts1, td1, gm1, exl1 = _prep(x, W1.astype(_f32), ms1, md1)
    pd1, po1 = _edge8(src, dst, ts1, td1, h1, gm1.reshape(16), z128, z16)
    h2, ts2, td2, gm2 = _mid(po1, pd1, exl1, h1,
                             b1.reshape(1, 128).astype(_f32),
                             W2.astype(_f32), ms2, md2)
    pd2, po2 = _edge1(src, dst, ts2, td2, h2, gm2.reshape(16), z128, z16)
    g = _final(po2, pd2, ts2, td2, gm2, h2,
               b2.reshape(1, 128).astype(_f32),
               batch.reshape(1, NN).astype(jnp.int32))
    return g
